# Initial kernel scaffold; baseline (speedup 1.0000x reference)
import jax
import jax.numpy as jnp
from jax.experimental import pallas as pl


def kernel(outputs, scores):
    raise NotImplementedError("write your pallas kernel here")


# speedup vs baseline: 2165.3663x; 2165.3663x over previous
import functools

import jax
import jax.numpy as jnp
from jax.experimental import pallas as pl

N = 256
CUTOFF = 10
LN2 = 0.6931471805599453


def _combine_kernel(o_col, o_row, s_col, s_row, out_ref):
    oc = o_col[...]
    orow = o_row[...]
    sc = s_col[...]
    srow = s_row[...]

    ii = jax.lax.broadcasted_iota(jnp.int32, (N, N), 0)
    kk = jax.lax.broadcasted_iota(jnp.int32, (N, N), 1)

    cmp_i = jnp.where(orow > oc, 1.0, 0.0) + jnp.where(
        (orow == oc) & (kk < ii), 1.0, 0.0)
    rank_col = jnp.sum(cmp_i, axis=1, keepdims=True)
    cmp_j = jnp.where(oc > orow, 1.0, 0.0) + jnp.where(
        (oc == orow) & (ii < kk), 1.0, 0.0)
    rank_row = jnp.sum(cmp_j, axis=0, keepdims=True)

    cmp_s = jnp.where(srow > sc, 1.0, 0.0) + jnp.where(
        (srow == sc) & (kk < ii), 1.0, 0.0)
    rank_s = jnp.sum(cmp_s, axis=1, keepdims=True)

    def disc(rank):
        return jnp.where(rank < CUTOFF,
                         LN2 / jnp.log(rank + 2.0), 0.0)

    d_col = disc(rank_col)
    d_row = disc(rank_row)
    d_s = disc(rank_s)

    g_col = jnp.exp(sc * LN2) - 1.0
    g_row = jnp.exp(srow * LN2) - 1.0
    idcg = jnp.sum(g_col * d_s, axis=(0, 1), keepdims=True)

    diff = oc - orow
    logits = jax.nn.sigmoid(diff)
    log_p = jnp.maximum(jnp.log(logits), -100.0)
    log_1mp = jnp.maximum(jnp.log(1.0 - logits), -100.0)
    labels = jnp.where(sc > srow, 1.0, 0.0)
    bce = -(labels * log_p + (1.0 - labels) * log_1mp)

    w = (jnp.abs(g_col - g_row) * jnp.abs(d_col - d_row)
         * jnp.where(oc != orow, 1.0, 0.0))
    total = jnp.sum(bce * w, axis=(0, 1), keepdims=True)
    out_ref[...] = total / (idcg * N)


def kernel(outputs, scores):
    o_col = outputs.reshape(N, 1)
    o_row = outputs.reshape(1, N)
    s_col = scores.reshape(N, 1)
    s_row = scores.reshape(1, N)
    loss = pl.pallas_call(
        _combine_kernel,
        out_shape=jax.ShapeDtypeStruct((1, 1), jnp.float32),
    )(o_col, o_row, s_col, s_row)
    return loss.reshape(())
